# baseline (speedup 1.0000x reference)
import functools

import jax
import jax.numpy as jnp
from jax import lax
from jax.experimental import pallas as pl
from jax.experimental.pallas import tpu as pltpu
from jax.experimental.pallas import tpu_sc as plsc

N_NODES = 10000
N_PAD = 10240
N_EDGES = 320000
N_WORKERS = 32
ROWS_PER_TILE = N_PAD // 16
BLK = 128
AGG_NB = 84
PHASES = 3
PH_NB = AGG_NB // PHASES
PH_E = PH_NB * BLK
EPW = AGG_NB * BLK
E_PAD = N_WORKERS * EPW
DEG_NB = AGG_NB

_MESH = plsc.VectorSubcoreMesh(core_axis_name="c", subcore_axis_name="s")


def _full16(v):
    return jnp.full((16,), v, dtype=jnp.int32)


_GDN = lax.GatherDimensionNumbers(
    offset_dims=(), collapsed_slice_dims=(0,), start_index_map=(0,))


def _bcast16(vec, i):
    return lax.gather(vec, _full16(i)[:, None], _GDN, (1,),
                      mode=lax.GatherScatterMode.PROMISE_IN_BOUNDS)


@functools.partial(
    pl.kernel,
    out_type=jax.ShapeDtypeStruct((2, N_PAD), jnp.float32),
    mesh=_MESH,
    compiler_params=pltpu.CompilerParams(needs_layout_passes=False),
    scratch_types=[
        pltpu.VMEM((DEG_NB, BLK), jnp.int32),
        pltpu.VMEM((DEG_NB, BLK), jnp.float32),
        pltpu.VMEM((ROWS_PER_TILE,), jnp.float32),
        pltpu.VMEM_SHARED((N_PAD,), jnp.float32),
    ],
)
def _deg_kernel(col_hbm, ew_hbm, deg_hbm, cidx, ewv, zbuf, deg_sh):
    cid = lax.axis_index("c")
    sid = lax.axis_index("s")
    wid = sid * 2 + cid
    z = jnp.zeros((16,), jnp.float32)
    for i in range(ROWS_PER_TILE // 16):
        zbuf[pl.ds(16 * i, 16)] = z
    pltpu.sync_copy(zbuf, deg_sh.at[pl.ds(sid * ROWS_PER_TILE, ROWS_PER_TILE)])
    plsc.subcore_barrier()
    pltpu.sync_copy(col_hbm.at[wid], cidx)
    pltpu.sync_copy(ew_hbm.at[wid], ewv)

    @pl.loop(0, DEG_NB)
    def _(j):
        pltpu.sync_copy(ewv.at[j], deg_sh.at[cidx.at[j]], add=True)

    plsc.subcore_barrier()
    sl = pl.ds(sid * ROWS_PER_TILE, ROWS_PER_TILE)
    pltpu.sync_copy(deg_sh.at[sl], deg_hbm.at[cid, sl])


def _make_agg_kernel(D, BLK=BLK, PH_NB=PH_NB):
    @functools.partial(
        pl.kernel,
        out_type=[jax.ShapeDtypeStruct((N_PAD, D), jnp.float32),
                  jax.ShapeDtypeStruct((N_PAD, D), jnp.float32)],
        mesh=_MESH,
        compiler_params=pltpu.CompilerParams(needs_layout_passes=False,
                                             use_tc_tiling_on_sc=False),
        scratch_types=[
            pltpu.VMEM((PH_E,), jnp.int32),
            pltpu.VMEM((PH_E,), jnp.int32),
            pltpu.VMEM((PH_E,), jnp.float32),
            pltpu.VMEM((PH_E,), jnp.float32),
            pltpu.VMEM((BLK,), jnp.int32),
            pltpu.VMEM((BLK,), jnp.int32),
            pltpu.VMEM((BLK,), jnp.int32),
            pltpu.VMEM((BLK,), jnp.int32),
            pltpu.VMEM((BLK, D), jnp.float32),
            pltpu.VMEM((BLK, D), jnp.float32),
            pltpu.VMEM_SHARED((N_PAD, D), jnp.float32),
            pltpu.SemaphoreType.DMA,
            pltpu.SemaphoreType.DMA,
            pltpu.SemaphoreType.DMA,
            pltpu.SemaphoreType.DMA,
            pltpu.SemaphoreType.DMA,
            pltpu.SemaphoreType.DMA,
        ],
    )
    def _agg(g_hbm, pair_hbm, ew_hbm, out0_hbm, out1_hbm,
             pairsA, pairsB, ewA, ewB, ri0, ri1, ci0, ci1,
             rows0, rows1, acc_sh, gs0, gs1, ss0, ss1, st0, st1):
        rows = (rows0, rows1)
        ridx = (ri0, ri1)
        cidx = (ci0, ci1)
        gsem = (gs0, gs1)
        ssem = (ss0, ss1)
        pairs = (pairsA, pairsB)
        ews = (ewA, ewB)
        stsem = (st0, st1)
        cid = lax.axis_index("c")
        sid = lax.axis_index("s")
        wid = sid * 2 + cid
        z = jnp.zeros((16,), jnp.float32)

        @pl.loop(0, BLK)
        def _(i):
            for f in range(D // 16):
                rows0[i, pl.ds(16 * f, 16)] = z

        base = sid * ROWS_PER_TILE
        for k in range(ROWS_PER_TILE // BLK):
            pltpu.sync_copy(rows0, acc_sh.at[pl.ds(base + k * BLK, BLK)])
        plsc.subcore_barrier()

        def stage(p, pb, start):
            c1 = start(pair_hbm.at[wid, pl.ds(p * PH_E, PH_E)],
                       pairs[pb], stsem[pb])
            c2 = start(ew_hbm.at[wid, pl.ds(p * PH_E, PH_E)],
                       ews[pb], stsem[pb])
            return c1, c2

        def unpack(pb, m, b):
            for t in range(BLK // 16):
                sl = pl.ds(16 * t, 16)
                pk = pairs[pb][pl.ds(m * BLK + 16 * t, 16)]
                ridx[b][sl] = lax.shift_right_logical(pk, 14)
                cidx[b][sl] = lax.bitwise_and(pk, 16383)

        stage(0, 0, pltpu.async_copy)
        for p in range(PHASES):
            pb = p % 2
            for c in stage(p, pb, pltpu.make_async_copy):
                c.wait()
            if p + 1 < PHASES:
                stage(p + 1, 1 - pb, pltpu.async_copy)

            unpack(pb, 0, 0)
            pltpu.async_copy(g_hbm.at[ri0], rows0, gs0)

            @pl.loop(0, PH_NB, step=2)
            def _(j):
                for b in range(2):
                    jb = j + b
                    bo = 1 - b

                    pltpu.make_async_copy(
                        g_hbm.at[ridx[b]], rows[b], gsem[b]).wait()

                    @pl.when(jb >= 1)
                    def _():
                        pltpu.make_async_copy(
                            rows[bo], acc_sh.at[cidx[bo]], ssem[bo]).wait()

                    @pl.when(jb + 1 < PH_NB)
                    def _():
                        unpack(pb, jb + 1, bo)
                        pltpu.async_copy(
                            g_hbm.at[ridx[bo]], rows[bo], gsem[bo])

                    @pl.loop(0, BLK // 16)
                    def _(gi):
                        ew16 = ews[pb][pl.ds(jb * BLK + 16 * gi, 16)]
                        for i in range(16):
                            bw = _bcast16(ew16, i)
                            e = gi * 16 + i
                            for f in range(D // 16):
                                sl = pl.ds(16 * f, 16)
                                rows[b][e, sl] = rows[b][e, sl] * bw

                    pltpu.async_copy(
                        rows[b], acc_sh.at[cidx[b]], ssem[b], add=True)

            pltpu.make_async_copy(
                rows[(PH_NB - 1) % 2], acc_sh.at[cidx[(PH_NB - 1) % 2]],
                ssem[(PH_NB - 1) % 2]).wait()

        plsc.subcore_barrier()
        sl = pl.ds(sid * ROWS_PER_TILE, ROWS_PER_TILE)

        @pl.when(cid == 0)
        def _():
            pltpu.sync_copy(acc_sh.at[sl], out0_hbm.at[sl])

        @pl.when(cid == 1)
        def _():
            pltpu.sync_copy(acc_sh.at[sl], out1_hbm.at[sl])

    return _agg


_agg128 = _make_agg_kernel(128, BLK=64, PH_NB=PH_E // 64)
_agg64 = _make_agg_kernel(64, BLK=64, PH_NB=PH_E // 64)


def _mm1_body(x_ref, w_ref, h_ref):
    h_ref[...] = jnp.dot(x_ref[...], w_ref[...],
                         preferred_element_type=jnp.float32)


def _scale_body(h_ref, d0_ref, d1_ref, g_ref, dinv_ref):
    dinv = lax.rsqrt(d0_ref[...] + d1_ref[...] + 1.0)
    g_ref[...] = h_ref[...] * dinv
    dinv_ref[...] = dinv


def _mm2_body(sa_ref, sb_ref, g0_ref, dinv_ref, b_ref, w_ref, g1_ref):
    dinv = dinv_ref[...]
    h = jax.nn.relu((sa_ref[...] + sb_ref[...] + g0_ref[...]) * dinv
                    + b_ref[...])
    g1_ref[...] = jnp.dot(h, w_ref[...],
                          preferred_element_type=jnp.float32) * dinv


def _fin_body(sa_ref, sb_ref, g1_ref, dinv_ref, bmu_ref, blv_ref,
              mu_ref, lv_ref):
    res = (sa_ref[...] + sb_ref[...] + g1_ref[...]) * dinv_ref[...]
    mu_ref[...] = res[:, :32] + bmu_ref[...]
    lv_ref[...] = res[:, 32:] + blv_ref[...]


_RB = 1024


def _rows_spec(d):
    return pl.BlockSpec((_RB, d), lambda i: (i, 0))


def _full_spec(a, b):
    return pl.BlockSpec((a, b), lambda i: (0, 0))


def kernel(x, edge_index, edge_weight, W1, b1, Wmu, bmu, Wlv, blv):
    row = edge_index[0].astype(jnp.int32)
    col = edge_index[1].astype(jnp.int32)
    ew = edge_weight.astype(jnp.float32)

    pad = E_PAD - N_EDGES
    fake = (jnp.arange(pad, dtype=jnp.int32) * 7) % N_NODES
    rowp = jnp.concatenate([row, fake])
    colp = jnp.concatenate([col, fake])
    ewp = jnp.concatenate([ew, jnp.zeros((pad,), jnp.float32)])
    colD = colp.reshape(N_WORKERS, DEG_NB, BLK)
    ewD = ewp.reshape(N_WORKERS, DEG_NB, BLK)
    pairA = ((rowp << 14) | colp).reshape(N_WORKERS, EPW)
    ewA = ewp.reshape(N_WORKERS, EPW)

    xp = jnp.pad(x, ((0, N_PAD - N_NODES), (0, 0)))
    Wcat = jnp.concatenate([Wmu, Wlv], axis=1)

    deg2 = _deg_kernel(colD, ewD)
    d0 = deg2[0][:, None]
    d1 = deg2[1][:, None]

    h0 = pl.pallas_call(
        _mm1_body,
        grid=(N_PAD // _RB,),
        in_specs=[_rows_spec(128), _full_spec(128, 128)],
        out_specs=_rows_spec(128),
        out_shape=jax.ShapeDtypeStruct((N_PAD, 128), jnp.float32),
    )(xp, W1)

    g0, dinv = pl.pallas_call(
        _scale_body,
        grid=(N_PAD // _RB,),
        in_specs=[_rows_spec(128), _rows_spec(1), _rows_spec(1)],
        out_specs=[_rows_spec(128), _rows_spec(1)],
        out_shape=[jax.ShapeDtypeStruct((N_PAD, 128), jnp.float32),
                   jax.ShapeDtypeStruct((N_PAD, 1), jnp.float32)],
    )(h0, d0, d1)

    s1a, s1b = _agg128(g0, pairA, ewA)

    g1 = pl.pallas_call(
        _mm2_body,
        grid=(N_PAD // _RB,),
        in_specs=[_rows_spec(128), _rows_spec(128), _rows_spec(128),
                  _rows_spec(1), _full_spec(1, 128), _full_spec(128, 64)],
        out_specs=_rows_spec(64),
        out_shape=jax.ShapeDtypeStruct((N_PAD, 64), jnp.float32),
    )(s1a, s1b, g0, dinv, b1[None, :], Wcat)

    s2a, s2b = _agg64(g1, pairA, ewA)

    mu, lv = pl.pallas_call(
        _fin_body,
        grid=(N_PAD // _RB,),
        in_specs=[_rows_spec(64), _rows_spec(64), _rows_spec(64),
                  _rows_spec(1), _full_spec(1, 32), _full_spec(1, 32)],
        out_specs=[_rows_spec(32), _rows_spec(32)],
        out_shape=[jax.ShapeDtypeStruct((N_PAD, 32), jnp.float32),
                   jax.ShapeDtypeStruct((N_PAD, 32), jnp.float32)],
    )(s2a, s2b, g1, dinv, bmu[None, :], blv[None, :])

    return (mu[:N_NODES], lv[:N_NODES])


# speedup vs baseline: 1.1009x; 1.1009x over previous
import functools

import jax
import jax.numpy as jnp
from jax import lax
from jax.experimental import pallas as pl
from jax.experimental.pallas import tpu as pltpu
from jax.experimental.pallas import tpu_sc as plsc

N_NODES = 10000
N_PAD = 10240
N_EDGES = 320000
N_WORKERS = 32
ROWS_PER_TILE = N_PAD // 16
BLK = 128
AGG_NB = 84
PHASES = 3
PH_NB = AGG_NB // PHASES
PH_E = PH_NB * BLK
EPW = AGG_NB * BLK
E_PAD = N_WORKERS * EPW
DEG_NB = AGG_NB

_MESH = plsc.VectorSubcoreMesh(core_axis_name="c", subcore_axis_name="s")


def _full16(v):
    return jnp.full((16,), v, dtype=jnp.int32)


_GDN = lax.GatherDimensionNumbers(
    offset_dims=(), collapsed_slice_dims=(0,), start_index_map=(0,))


def _bcast16(vec, i):
    return lax.gather(vec, _full16(i)[:, None], _GDN, (1,),
                      mode=lax.GatherScatterMode.PROMISE_IN_BOUNDS)


@functools.partial(
    pl.kernel,
    out_type=jax.ShapeDtypeStruct((2, N_PAD), jnp.float32),
    mesh=_MESH,
    compiler_params=pltpu.CompilerParams(needs_layout_passes=False),
    scratch_types=[
        pltpu.VMEM((DEG_NB, BLK), jnp.int32),
        pltpu.VMEM((DEG_NB, BLK), jnp.float32),
        pltpu.VMEM((ROWS_PER_TILE,), jnp.float32),
        pltpu.VMEM_SHARED((N_PAD,), jnp.float32),
    ],
)
def _deg_kernel(col_hbm, ew_hbm, deg_hbm, cidx, ewv, zbuf, deg_sh):
    cid = lax.axis_index("c")
    sid = lax.axis_index("s")
    wid = sid * 2 + cid
    z = jnp.zeros((16,), jnp.float32)
    for i in range(ROWS_PER_TILE // 16):
        zbuf[pl.ds(16 * i, 16)] = z
    pltpu.sync_copy(zbuf, deg_sh.at[pl.ds(sid * ROWS_PER_TILE, ROWS_PER_TILE)])
    plsc.subcore_barrier()
    pltpu.sync_copy(col_hbm.at[wid], cidx)
    pltpu.sync_copy(ew_hbm.at[wid], ewv)

    @pl.loop(0, DEG_NB)
    def _(j):
        pltpu.sync_copy(ewv.at[j], deg_sh.at[cidx.at[j]], add=True)

    plsc.subcore_barrier()
    sl = pl.ds(sid * ROWS_PER_TILE, ROWS_PER_TILE)
    pltpu.sync_copy(deg_sh.at[sl], deg_hbm.at[cid, sl])


def _make_agg_kernel(D, BLK=BLK, PH_NB=PH_NB):
    @functools.partial(
        pl.kernel,
        out_type=[jax.ShapeDtypeStruct((N_PAD, D), jnp.float32),
                  jax.ShapeDtypeStruct((N_PAD, D), jnp.float32)],
        mesh=_MESH,
        compiler_params=pltpu.CompilerParams(needs_layout_passes=False,
                                             use_tc_tiling_on_sc=False),
        scratch_types=[
            pltpu.VMEM((PH_E,), jnp.int32),
            pltpu.VMEM((PH_E,), jnp.int32),
            pltpu.VMEM((PH_E,), jnp.float32),
            pltpu.VMEM((PH_E,), jnp.float32),
            pltpu.VMEM((BLK,), jnp.int32),
            pltpu.VMEM((BLK,), jnp.int32),
            pltpu.VMEM((BLK,), jnp.int32),
            pltpu.VMEM((BLK,), jnp.int32),
            pltpu.VMEM((BLK, D), jnp.float32),
            pltpu.VMEM((BLK, D), jnp.float32),
            pltpu.VMEM_SHARED((N_PAD, D), jnp.float32),
            pltpu.SemaphoreType.DMA,
            pltpu.SemaphoreType.DMA,
            pltpu.SemaphoreType.DMA,
            pltpu.SemaphoreType.DMA,
            pltpu.SemaphoreType.DMA,
            pltpu.SemaphoreType.DMA,
        ],
    )
    def _agg(g_hbm, pair_hbm, ew_hbm, out0_hbm, out1_hbm,
             pairsA, pairsB, ewA, ewB, ri0, ri1, ci0, ci1,
             rows0, rows1, acc_sh, gs0, gs1, ss0, ss1, st0, st1):
        rows = (rows0, rows1)
        ridx = (ri0, ri1)
        cidx = (ci0, ci1)
        gsem = (gs0, gs1)
        ssem = (ss0, ss1)
        pairs = (pairsA, pairsB)
        ews = (ewA, ewB)
        stsem = (st0, st1)
        cid = lax.axis_index("c")
        sid = lax.axis_index("s")
        wid = sid * 2 + cid
        z = jnp.zeros((16,), jnp.float32)

        @pl.loop(0, BLK)
        def _(i):
            for f in range(D // 16):
                rows0[i, pl.ds(16 * f, 16)] = z

        base = sid * ROWS_PER_TILE
        for k in range(ROWS_PER_TILE // BLK):
            pltpu.sync_copy(rows0, acc_sh.at[pl.ds(base + k * BLK, BLK)])
        plsc.subcore_barrier()

        def stage(p, pb, start):
            c1 = start(pair_hbm.at[wid, pl.ds(p * PH_E, PH_E)],
                       pairs[pb], stsem[pb])
            c2 = start(ew_hbm.at[wid, pl.ds(p * PH_E, PH_E)],
                       ews[pb], stsem[pb])
            return c1, c2

        def unpack(pb, m, b):
            for t in range(BLK // 16):
                sl = pl.ds(16 * t, 16)
                pk = pairs[pb][pl.ds(m * BLK + 16 * t, 16)]
                ridx[b][sl] = lax.shift_right_logical(pk, 14)
                cidx[b][sl] = lax.bitwise_and(pk, 16383)

        stage(0, 0, pltpu.async_copy)
        for p in range(PHASES):
            pb = p % 2
            for c in stage(p, pb, pltpu.make_async_copy):
                c.wait()
            if p + 1 < PHASES:
                stage(p + 1, 1 - pb, pltpu.async_copy)

            unpack(pb, 0, 0)
            pltpu.async_copy(g_hbm.at[ri0], rows0, gs0)

            @pl.loop(0, PH_NB, step=2)
            def _(j):
                for b in range(2):
                    jb = j + b
                    bo = 1 - b

                    pltpu.make_async_copy(
                        g_hbm.at[ridx[b]], rows[b], gsem[b]).wait()

                    @pl.when(jb >= 1)
                    def _():
                        pltpu.make_async_copy(
                            rows[bo], acc_sh.at[cidx[bo]], ssem[bo]).wait()

                    @pl.when(jb + 1 < PH_NB)
                    def _():
                        unpack(pb, jb + 1, bo)
                        pltpu.async_copy(
                            g_hbm.at[ridx[bo]], rows[bo], gsem[bo])

                    @pl.loop(0, BLK // 16)
                    def _(gi):
                        ew16 = ews[pb][pl.ds(jb * BLK + 16 * gi, 16)]
                        for i in range(16):
                            bw = _bcast16(ew16, i)
                            e = gi * 16 + i
                            for f in range(D // 16):
                                sl = pl.ds(16 * f, 16)
                                rows[b][e, sl] = rows[b][e, sl] * bw

                    pltpu.async_copy(
                        rows[b], acc_sh.at[cidx[b]], ssem[b], add=True)

            pltpu.make_async_copy(
                rows[(PH_NB - 1) % 2], acc_sh.at[cidx[(PH_NB - 1) % 2]],
                ssem[(PH_NB - 1) % 2]).wait()

        plsc.subcore_barrier()
        sl = pl.ds(sid * ROWS_PER_TILE, ROWS_PER_TILE)

        @pl.when(cid == 0)
        def _():
            pltpu.sync_copy(acc_sh.at[sl], out0_hbm.at[sl])

        @pl.when(cid == 1)
        def _():
            pltpu.sync_copy(acc_sh.at[sl], out1_hbm.at[sl])

    return _agg


_agg128 = _make_agg_kernel(128)
_agg64 = _make_agg_kernel(64)


def _mm1_body(x_ref, w_ref, h_ref):
    h_ref[...] = jnp.dot(x_ref[...], w_ref[...],
                         preferred_element_type=jnp.float32)


def _scale_body(h_ref, d0_ref, d1_ref, g_ref, dinv_ref):
    dinv = lax.rsqrt(d0_ref[...] + d1_ref[...] + 1.0)
    g_ref[...] = h_ref[...] * dinv
    dinv_ref[...] = dinv


def _mm2_body(sa_ref, sb_ref, g0_ref, dinv_ref, b_ref, w_ref, g1_ref):
    dinv = dinv_ref[...]
    h = jax.nn.relu((sa_ref[...] + sb_ref[...] + g0_ref[...]) * dinv
                    + b_ref[...])
    g1_ref[...] = jnp.dot(h, w_ref[...],
                          preferred_element_type=jnp.float32) * dinv


def _fin_body(sa_ref, sb_ref, g1_ref, dinv_ref, bmu_ref, blv_ref,
              mu_ref, lv_ref):
    res = (sa_ref[...] + sb_ref[...] + g1_ref[...]) * dinv_ref[...]
    mu_ref[...] = res[:, :32] + bmu_ref[...]
    lv_ref[...] = res[:, 32:] + blv_ref[...]


_RB = 1024


def _rows_spec(d):
    return pl.BlockSpec((_RB, d), lambda i: (i, 0))


def _full_spec(a, b):
    return pl.BlockSpec((a, b), lambda i: (0, 0))


def kernel(x, edge_index, edge_weight, W1, b1, Wmu, bmu, Wlv, blv):
    row = edge_index[0].astype(jnp.int32)
    col = edge_index[1].astype(jnp.int32)
    ew = edge_weight.astype(jnp.float32)

    pad = E_PAD - N_EDGES
    fake = (jnp.arange(pad, dtype=jnp.int32) * 7) % N_NODES
    rowp = jnp.concatenate([row, fake])
    colp = jnp.concatenate([col, fake])
    ewp = jnp.concatenate([ew, jnp.zeros((pad,), jnp.float32)])
    colD = colp.reshape(N_WORKERS, DEG_NB, BLK)
    ewD = ewp.reshape(N_WORKERS, DEG_NB, BLK)
    pairA = ((rowp << 14) | colp).reshape(N_WORKERS, EPW)
    ewA = ewp.reshape(N_WORKERS, EPW)

    xp = jnp.pad(x, ((0, N_PAD - N_NODES), (0, 0)))
    Wcat = jnp.concatenate([Wmu, Wlv], axis=1)

    deg2 = _deg_kernel(colD, ewD)
    d0 = deg2[0][:, None]
    d1 = deg2[1][:, None]

    h0 = pl.pallas_call(
        _mm1_body,
        grid=(N_PAD // _RB,),
        in_specs=[_rows_spec(128), _full_spec(128, 128)],
        out_specs=_rows_spec(128),
        out_shape=jax.ShapeDtypeStruct((N_PAD, 128), jnp.float32),
    )(xp, W1)

    g0, dinv = pl.pallas_call(
        _scale_body,
        grid=(N_PAD // _RB,),
        in_specs=[_rows_spec(128), _rows_spec(1), _rows_spec(1)],
        out_specs=[_rows_spec(128), _rows_spec(1)],
        out_shape=[jax.ShapeDtypeStruct((N_PAD, 128), jnp.float32),
                   jax.ShapeDtypeStruct((N_PAD, 1), jnp.float32)],
    )(h0, d0, d1)

    s1a, s1b = _agg128(g0, pairA, ewA)

    g1 = pl.pallas_call(
        _mm2_body,
        grid=(N_PAD // _RB,),
        in_specs=[_rows_spec(128), _rows_spec(128), _rows_spec(128),
                  _rows_spec(1), _full_spec(1, 128), _full_spec(128, 64)],
        out_specs=_rows_spec(64),
        out_shape=jax.ShapeDtypeStruct((N_PAD, 64), jnp.float32),
    )(s1a, s1b, g0, dinv, b1[None, :], Wcat)

    s2a, s2b = _agg64(g1, pairA, ewA)

    mu, lv = pl.pallas_call(
        _fin_body,
        grid=(N_PAD // _RB,),
        in_specs=[_rows_spec(64), _rows_spec(64), _rows_spec(64),
                  _rows_spec(1), _full_spec(1, 32), _full_spec(1, 32)],
        out_specs=[_rows_spec(32), _rows_spec(32)],
        out_shape=[jax.ShapeDtypeStruct((N_PAD, 32), jnp.float32),
                   jax.ShapeDtypeStruct((N_PAD, 32), jnp.float32)],
    )(s2a, s2b, g1, dinv, bmu[None, :], blv[None, :])

    return (mu[:N_NODES], lv[:N_NODES])


# speedup vs baseline: 1.3467x; 1.2233x over previous
import functools

import jax
import jax.numpy as jnp
from jax import lax
from jax.experimental import pallas as pl
from jax.experimental.pallas import tpu as pltpu
from jax.experimental.pallas import tpu_sc as plsc

N_NODES = 10000
N_PAD = 10240
N_EDGES = 320000
N_WORKERS = 32
ROWS_PER_TILE = N_PAD // 16
BLK = 128
AGG_NB = 84
PHASES = 3
PH_NB = AGG_NB // PHASES
PH_E = PH_NB * BLK
EPW = AGG_NB * BLK
E_PAD = N_WORKERS * EPW
DEG_NB = AGG_NB

_MESH = plsc.VectorSubcoreMesh(core_axis_name="c", subcore_axis_name="s")


def _full16(v):
    return jnp.full((16,), v, dtype=jnp.int32)


_GDN = lax.GatherDimensionNumbers(
    offset_dims=(), collapsed_slice_dims=(0,), start_index_map=(0,))


def _bcast16(vec, i):
    return lax.gather(vec, _full16(i)[:, None], _GDN, (1,),
                      mode=lax.GatherScatterMode.PROMISE_IN_BOUNDS)


@functools.partial(
    pl.kernel,
    out_type=jax.ShapeDtypeStruct((2, N_PAD), jnp.float32),
    mesh=_MESH,
    compiler_params=pltpu.CompilerParams(needs_layout_passes=False),
    scratch_types=[
        pltpu.VMEM((DEG_NB, BLK), jnp.int32),
        pltpu.VMEM((DEG_NB, BLK), jnp.float32),
        pltpu.VMEM((ROWS_PER_TILE,), jnp.float32),
        pltpu.VMEM_SHARED((N_PAD,), jnp.float32),
    ],
)
def _deg_kernel(col_hbm, ew_hbm, deg_hbm, cidx, ewv, zbuf, deg_sh):
    cid = lax.axis_index("c")
    sid = lax.axis_index("s")
    wid = sid * 2 + cid
    z = jnp.zeros((16,), jnp.float32)
    for i in range(ROWS_PER_TILE // 16):
        zbuf[pl.ds(16 * i, 16)] = z
    pltpu.sync_copy(zbuf, deg_sh.at[pl.ds(sid * ROWS_PER_TILE, ROWS_PER_TILE)])
    plsc.subcore_barrier()
    pltpu.sync_copy(col_hbm.at[wid], cidx)
    pltpu.sync_copy(ew_hbm.at[wid], ewv)

    @pl.loop(0, DEG_NB)
    def _(j):
        pltpu.sync_copy(ewv.at[j], deg_sh.at[cidx.at[j]], add=True)

    plsc.subcore_barrier()
    sl = pl.ds(sid * ROWS_PER_TILE, ROWS_PER_TILE)
    pltpu.sync_copy(deg_sh.at[sl], deg_hbm.at[cid, sl])


def _make_agg_kernel(D, BLK=BLK, PH_NB=PH_NB):
    @functools.partial(
        pl.kernel,
        out_type=[jax.ShapeDtypeStruct((N_PAD, D), jnp.float32),
                  jax.ShapeDtypeStruct((N_PAD, D), jnp.float32)],
        mesh=_MESH,
        compiler_params=pltpu.CompilerParams(needs_layout_passes=False,
                                             use_tc_tiling_on_sc=False),
        scratch_types=[
            pltpu.VMEM((PH_E,), jnp.int32),
            pltpu.VMEM((PH_E,), jnp.int32),
            pltpu.VMEM((PH_E,), jnp.float32),
            pltpu.VMEM((PH_E,), jnp.float32),
            pltpu.VMEM((BLK,), jnp.int32),
            pltpu.VMEM((BLK,), jnp.int32),
            pltpu.VMEM((BLK,), jnp.int32),
            pltpu.VMEM((BLK,), jnp.int32),
            pltpu.VMEM((BLK, D), jnp.float32),
            pltpu.VMEM((BLK, D), jnp.float32),
            pltpu.VMEM_SHARED((N_PAD, D), jnp.float32),
            pltpu.SemaphoreType.DMA,
            pltpu.SemaphoreType.DMA,
            pltpu.SemaphoreType.DMA,
            pltpu.SemaphoreType.DMA,
            pltpu.SemaphoreType.DMA,
            pltpu.SemaphoreType.DMA,
        ],
    )
    def _agg(g_hbm, pair_hbm, ew_hbm, out0_hbm, out1_hbm,
             pairsA, pairsB, ewA, ewB, ri0, ri1, ci0, ci1,
             rows0, rows1, acc_sh, gs0, gs1, ss0, ss1, st0, st1):
        rows = (rows0, rows1)
        ridx = (ri0, ri1)
        cidx = (ci0, ci1)
        gsem = (gs0, gs1)
        ssem = (ss0, ss1)
        pairs = (pairsA, pairsB)
        ews = (ewA, ewB)
        stsem = (st0, st1)
        cid = lax.axis_index("c")
        sid = lax.axis_index("s")
        wid = sid * 2 + cid
        z = jnp.zeros((16,), jnp.float32)

        @pl.loop(0, BLK)
        def _(i):
            for f in range(D // 16):
                rows0[i, pl.ds(16 * f, 16)] = z

        base = sid * ROWS_PER_TILE
        for k in range(ROWS_PER_TILE // BLK):
            pltpu.sync_copy(rows0, acc_sh.at[pl.ds(base + k * BLK, BLK)])
        plsc.subcore_barrier()

        def stage(p, pb, start):
            c1 = start(pair_hbm.at[wid, pl.ds(p * PH_E, PH_E)],
                       pairs[pb], stsem[pb])
            c2 = start(ew_hbm.at[wid, pl.ds(p * PH_E, PH_E)],
                       ews[pb], stsem[pb])
            return c1, c2

        def unpack(pb, m, b):
            for t in range(BLK // 16):
                sl = pl.ds(16 * t, 16)
                pk = pairs[pb][pl.ds(m * BLK + 16 * t, 16)]
                ridx[b][sl] = lax.shift_right_logical(pk, 14)
                cidx[b][sl] = lax.bitwise_and(pk, 16383)

        stage(0, 0, pltpu.async_copy)
        for p in range(PHASES):
            pb = p % 2
            for c in stage(p, pb, pltpu.make_async_copy):
                c.wait()
            if p + 1 < PHASES:
                stage(p + 1, 1 - pb, pltpu.async_copy)

            unpack(pb, 0, 0)
            pltpu.async_copy(g_hbm.at[ri0], rows0, gs0)

            @pl.loop(0, PH_NB, step=2)
            def _(j):
                for b in range(2):
                    jb = j + b
                    bo = 1 - b

                    pltpu.make_async_copy(
                        g_hbm.at[ridx[b]], rows[b], gsem[b]).wait()

                    @pl.when(jb >= 1)
                    def _():
                        pltpu.make_async_copy(
                            rows[bo], acc_sh.at[cidx[bo]], ssem[bo]).wait()

                    @pl.when(jb + 1 < PH_NB)
                    def _():
                        unpack(pb, jb + 1, bo)
                        pltpu.async_copy(
                            g_hbm.at[ridx[bo]], rows[bo], gsem[bo])

                    @pl.loop(0, BLK // 16)
                    def _(gi):
                        ew16 = ews[pb][pl.ds(jb * BLK + 16 * gi, 16)]
                        for i in range(16):
                            bw = _bcast16(ew16, i)
                            e = gi * 16 + i
                            for f in range(D // 16):
                                sl = pl.ds(16 * f, 16)
                                rows[b][e, sl] = rows[b][e, sl] * bw

                    pltpu.async_copy(
                        rows[b], acc_sh.at[cidx[b]], ssem[b], add=True)

            pltpu.make_async_copy(
                rows[(PH_NB - 1) % 2], acc_sh.at[cidx[(PH_NB - 1) % 2]],
                ssem[(PH_NB - 1) % 2]).wait()

        plsc.subcore_barrier()
        sl = pl.ds(sid * ROWS_PER_TILE, ROWS_PER_TILE)

        @pl.when(cid == 0)
        def _():
            pltpu.sync_copy(acc_sh.at[sl], out0_hbm.at[sl])

        @pl.when(cid == 1)
        def _():
            pltpu.sync_copy(acc_sh.at[sl], out1_hbm.at[sl])

    return _agg


_agg128 = _make_agg_kernel(128)


def _mm1_body(x_ref, w_ref, h_ref):
    h_ref[...] = jnp.dot(x_ref[...], w_ref[...],
                         preferred_element_type=jnp.float32)


def _scale_body(h_ref, d0_ref, d1_ref, g_ref, dinv_ref):
    dinv = lax.rsqrt(d0_ref[...] + d1_ref[...] + 1.0)
    g_ref[...] = h_ref[...] * dinv
    dinv_ref[...] = dinv


def _mm2_body(sa_ref, sb_ref, g0_ref, dinv_ref, b_ref, w_ref, g1_ref):
    dinv = dinv_ref[...]
    h = jax.nn.relu((sa_ref[...] + sb_ref[...] + g0_ref[...]) * dinv
                    + b_ref[...])
    g1_ref[...] = jnp.dot(h, w_ref[...],
                          preferred_element_type=jnp.float32) * dinv


def _fin_body(sa_ref, sb_ref, g1_ref, dinv_ref, bmu_ref, blv_ref,
              mu_ref, lv_ref):
    res = (sa_ref[...] + sb_ref[...] + g1_ref[...]) * dinv_ref[...]
    mu_ref[...] = res[:, :32] + bmu_ref[...]
    lv_ref[...] = res[:, 32:64] + blv_ref[...]


_RB = 1024


def _rows_spec(d):
    return pl.BlockSpec((_RB, d), lambda i: (i, 0))


def _full_spec(a, b):
    return pl.BlockSpec((a, b), lambda i: (0, 0))


def kernel(x, edge_index, edge_weight, W1, b1, Wmu, bmu, Wlv, blv):
    row = edge_index[0].astype(jnp.int32)
    col = edge_index[1].astype(jnp.int32)
    ew = edge_weight.astype(jnp.float32)

    pad = E_PAD - N_EDGES
    fake = (jnp.arange(pad, dtype=jnp.int32) * 7) % N_NODES
    rowp = jnp.concatenate([row, fake])
    colp = jnp.concatenate([col, fake])
    ewp = jnp.concatenate([ew, jnp.zeros((pad,), jnp.float32)])
    colD = colp.reshape(N_WORKERS, DEG_NB, BLK)
    ewD = ewp.reshape(N_WORKERS, DEG_NB, BLK)
    pairA = ((rowp << 14) | colp).reshape(N_WORKERS, EPW)
    ewA = ewp.reshape(N_WORKERS, EPW)

    xp = jnp.pad(x, ((0, N_PAD - N_NODES), (0, 0)))
    Wcat = jnp.pad(jnp.concatenate([Wmu, Wlv], axis=1), ((0, 0), (0, 64)))

    deg2 = _deg_kernel(colD, ewD)
    d0 = deg2[0][:, None]
    d1 = deg2[1][:, None]

    h0 = pl.pallas_call(
        _mm1_body,
        grid=(N_PAD // _RB,),
        in_specs=[_rows_spec(128), _full_spec(128, 128)],
        out_specs=_rows_spec(128),
        out_shape=jax.ShapeDtypeStruct((N_PAD, 128), jnp.float32),
    )(xp, W1)

    g0, dinv = pl.pallas_call(
        _scale_body,
        grid=(N_PAD // _RB,),
        in_specs=[_rows_spec(128), _rows_spec(1), _rows_spec(1)],
        out_specs=[_rows_spec(128), _rows_spec(1)],
        out_shape=[jax.ShapeDtypeStruct((N_PAD, 128), jnp.float32),
                   jax.ShapeDtypeStruct((N_PAD, 1), jnp.float32)],
    )(h0, d0, d1)

    s1a, s1b = _agg128(g0, pairA, ewA)

    g1 = pl.pallas_call(
        _mm2_body,
        grid=(N_PAD // _RB,),
        in_specs=[_rows_spec(128), _rows_spec(128), _rows_spec(128),
                  _rows_spec(1), _full_spec(1, 128), _full_spec(128, 128)],
        out_specs=_rows_spec(128),
        out_shape=jax.ShapeDtypeStruct((N_PAD, 128), jnp.float32),
    )(s1a, s1b, g0, dinv, b1[None, :], Wcat)

    s2a, s2b = _agg128(g1, pairA, ewA)

    mu, lv = pl.pallas_call(
        _fin_body,
        grid=(N_PAD // _RB,),
        in_specs=[_rows_spec(128), _rows_spec(128), _rows_spec(128),
                  _rows_spec(1), _full_spec(1, 32), _full_spec(1, 32)],
        out_specs=[_rows_spec(32), _rows_spec(32)],
        out_shape=[jax.ShapeDtypeStruct((N_PAD, 32), jnp.float32),
                   jax.ShapeDtypeStruct((N_PAD, 32), jnp.float32)],
    )(s2a, s2b, g1, dinv, bmu[None, :], blv[None, :])

    return (mu[:N_NODES], lv[:N_NODES])
